# Initial kernel scaffold; baseline (speedup 1.0000x reference)
import jax
import jax.numpy as jnp
from jax.experimental import pallas as pl


def kernel(x, edge_index, W1, b1, W2, b2, W3, b3, Wc1, bc1, Wc2, bc2, Wc3, bc3):
    raise NotImplementedError("write your pallas kernel here")


# speedup vs baseline: 43.7637x; 43.7637x over previous
import functools

import jax
import jax.numpy as jnp
from jax import lax
from jax.experimental import pallas as pl
from jax.experimental.pallas import tpu as pltpu
from jax.experimental.pallas import tpu_sc as plsc

_NC = 2
_NS = 16
_NW = _NC * _NS
_L = 16

_MESH = plsc.VectorSubcoreMesh(core_axis_name="c", subcore_axis_name="s")
_SC_PARAMS = pltpu.CompilerParams(needs_layout_passes=False)


def _worker_id():
    return lax.axis_index("s") * _NC + lax.axis_index("c")


def _zero_vmem(ref, n):
    z = jnp.zeros((_L,), jnp.float32)

    def body(i, c):
        ref[pl.ds(i * _L, _L)] = z
        return c

    lax.fori_loop(0, n // _L, body, 0)


def _make_sc_degrees(n, e):
    epw = e // _NW
    assert e % (_NW * _L) == 0 and n % _L == 0

    @functools.partial(
        pl.kernel,
        mesh=_MESH,
        out_type=[
            jax.ShapeDtypeStruct((_NW, n), jnp.float32),
            jax.ShapeDtypeStruct((_NW, n), jnp.float32),
        ],
        scratch_types=[
            pltpu.VMEM((epw,), jnp.int32),
            pltpu.VMEM((epw,), jnp.int32),
            pltpu.VMEM((n,), jnp.float32),
            pltpu.VMEM((n,), jnp.float32),
        ],
        compiler_params=_SC_PARAMS,
    )
    def degrees(src_hbm, dst_hbm, out_o_hbm, out_i_hbm, src_v, dst_v, acc_o, acc_i):
        wid = _worker_id()
        base = wid * epw
        pltpu.sync_copy(src_hbm.at[pl.ds(base, epw)], src_v)
        pltpu.sync_copy(dst_hbm.at[pl.ds(base, epw)], dst_v)
        _zero_vmem(acc_o, n)
        _zero_vmem(acc_i, n)
        ones = jnp.ones((_L,), jnp.float32)

        def body(i, c):
            s = src_v[pl.ds(i * _L, _L)]
            d = dst_v[pl.ds(i * _L, _L)]
            plsc.addupdate_scatter(acc_o, [s], ones)
            plsc.addupdate_scatter(acc_i, [d], ones)
            return c

        lax.fori_loop(0, epw // _L, body, 0)
        pltpu.sync_copy(acc_o, out_o_hbm.at[wid])
        pltpu.sync_copy(acc_i, out_i_hbm.at[wid])

    return degrees


def _make_sc_bpass(n, e):
    epw = e // _NW

    @functools.partial(
        pl.kernel,
        mesh=_MESH,
        out_type=jax.ShapeDtypeStruct((_NW, n), jnp.float32),
        scratch_types=[
            pltpu.VMEM((epw,), jnp.int32),
            pltpu.VMEM((epw,), jnp.int32),
            pltpu.VMEM((n,), jnp.float32),
            pltpu.VMEM((n,), jnp.float32),
        ],
        compiler_params=_SC_PARAMS,
    )
    def bpass(src_hbm, dst_hbm, p_hbm, out_hbm, src_v, dst_v, p_v, acc):
        wid = _worker_id()
        base = wid * epw
        pltpu.sync_copy(src_hbm.at[pl.ds(base, epw)], src_v)
        pltpu.sync_copy(dst_hbm.at[pl.ds(base, epw)], dst_v)
        pltpu.sync_copy(p_hbm, p_v)
        _zero_vmem(acc, n)

        def body(i, c):
            d = dst_v[pl.ds(i * _L, _L)]
            vals = plsc.load_gather(p_v, [d])
            s = src_v[pl.ds(i * _L, _L)]
            plsc.addupdate_scatter(acc, [s], vals)
            return c

        lax.fori_loop(0, epw // _L, body, 0)
        pltpu.sync_copy(acc, out_hbm.at[wid])

    return bpass


def _tc_norms_body(po_ref, pi_ref, ns_ref, nd_ref, p_ref, inv_n):
    deg_o = jnp.sum(po_ref[...], axis=0, keepdims=True)
    deg_i = jnp.sum(pi_ref[...], axis=0, keepdims=True)
    ns = lax.rsqrt(jnp.maximum(deg_o, 1.0))
    nd = lax.rsqrt(jnp.maximum(deg_i, 1.0))
    ns_ref[...] = ns
    nd_ref[...] = nd
    p_ref[...] = nd * inv_n


def _tc_mid_body(part_ref, ns_ref, nd_ref, p_ref, sig_ref):
    a = jnp.sum(part_ref[...], axis=0, keepdims=True)
    v = ns_ref[...] * a
    sig_ref[...] = jnp.sum(v, axis=(0, 1), keepdims=True)
    p_ref[...] = nd_ref[...] * v


def _leaky(v):
    return jnp.where(v >= 0, v, 0.01 * v)


def _tc_head_body(
    part_ref, ns_ref, x_ref,
    w1_ref, b1_ref, w2_ref, b2_ref, w3_ref, b3_ref,
    wc1_ref, bc1_ref, wc2_ref, bc2_ref, wc3_ref, bc3_ref,
    s2_ref, s3_ref, y_ref,
):
    a = jnp.sum(part_ref[...], axis=0, keepdims=True)
    v1 = ns_ref[...] * a
    r = jnp.dot(v1, x_ref[...], preferred_element_type=jnp.float32)
    r = jnp.dot(r, w1_ref[...], preferred_element_type=jnp.float32) + s2_ref[...] * b1_ref[...]
    r = jnp.dot(r, w2_ref[...], preferred_element_type=jnp.float32) + s3_ref[...] * b2_ref[...]
    g = jnp.dot(r, w3_ref[...], preferred_element_type=jnp.float32) + b3_ref[...]
    y = _leaky(jnp.dot(g, wc1_ref[...], preferred_element_type=jnp.float32) + bc1_ref[...])
    y = _leaky(jnp.dot(y, wc2_ref[...], preferred_element_type=jnp.float32) + bc2_ref[...])
    y = _leaky(jnp.dot(y, wc3_ref[...], preferred_element_type=jnp.float32) + bc3_ref[...])
    y_ref[...] = y


def kernel(x, edge_index, W1, b1, W2, b2, W3, b3, Wc1, bc1, Wc2, bc2, Wc3, bc3):
    n, d_in = x.shape
    e = edge_index.shape[1]
    src = edge_index[0]
    dst = edge_index[1]

    sc_degrees = _make_sc_degrees(n, e)
    sc_bpass = _make_sc_bpass(n, e)

    po, pi = sc_degrees(src, dst)

    f32 = jnp.float32
    norms = pl.pallas_call(
        functools.partial(_tc_norms_body, inv_n=1.0 / n),
        out_shape=[
            jax.ShapeDtypeStruct((1, n), f32),
            jax.ShapeDtypeStruct((1, n), f32),
            jax.ShapeDtypeStruct((1, n), f32),
        ],
    )
    ns, nd, p = norms(po, pi)

    mid = pl.pallas_call(
        _tc_mid_body,
        out_shape=[
            jax.ShapeDtypeStruct((1, n), f32),
            jax.ShapeDtypeStruct((1, 1), f32),
        ],
    )

    part = sc_bpass(src, dst, p.reshape(n))
    p, s3 = mid(part, ns, nd)
    part = sc_bpass(src, dst, p.reshape(n))
    p, s2 = mid(part, ns, nd)
    part = sc_bpass(src, dst, p.reshape(n))

    head = pl.pallas_call(
        _tc_head_body,
        out_shape=jax.ShapeDtypeStruct((1, Wc3.shape[1]), f32),
    )
    y = head(
        part, ns, x,
        W1, b1.reshape(1, -1), W2, b2.reshape(1, -1), W3, b3.reshape(1, -1),
        Wc1, bc1.reshape(1, -1), Wc2, bc2.reshape(1, -1), Wc3, bc3.reshape(1, -1),
        s2, s3,
    )
    return y


# speedup vs baseline: 52.0428x; 1.1892x over previous
import functools

import jax
import jax.numpy as jnp
from jax import lax
from jax.experimental import pallas as pl
from jax.experimental.pallas import tpu as pltpu
from jax.experimental.pallas import tpu_sc as plsc

_NC = 2
_NS = 16
_NW = _NC * _NS
_L = 16

_MESH = plsc.VectorSubcoreMesh(core_axis_name="c", subcore_axis_name="s")
_SC_PARAMS = pltpu.CompilerParams(needs_layout_passes=False)


def _worker_id():
    return lax.axis_index("s") * _NC + lax.axis_index("c")


def _zero_vmem(ref, n):
    z = jnp.zeros((_L,), jnp.float32)

    def body(i, c):
        ref[pl.ds(i * _L, _L)] = z
        return c

    lax.fori_loop(0, n // _L, body, 0, unroll=8)


def _make_sc_degrees(n, e):
    epw = e // _NW
    assert e % (_NW * _L) == 0 and n % _L == 0

    @functools.partial(
        pl.kernel,
        mesh=_MESH,
        out_type=[
            jax.ShapeDtypeStruct((_NW, n), jnp.float32),
            jax.ShapeDtypeStruct((_NW, n), jnp.float32),
        ],
        scratch_types=[
            pltpu.VMEM((epw,), jnp.int32),
            pltpu.VMEM((epw,), jnp.int32),
            pltpu.VMEM((n,), jnp.float32),
            pltpu.VMEM((n,), jnp.float32),
            pltpu.SemaphoreType.DMA,
        ],
        compiler_params=_SC_PARAMS,
    )
    def degrees(src_hbm, dst_hbm, out_o_hbm, out_i_hbm, src_v, dst_v, acc_o, acc_i, sem):
        wid = _worker_id()
        base = wid * epw
        cp_s = pltpu.async_copy(src_hbm.at[pl.ds(base, epw)], src_v, sem)
        cp_d = pltpu.async_copy(dst_hbm.at[pl.ds(base, epw)], dst_v, sem)
        _zero_vmem(acc_o, n)
        _zero_vmem(acc_i, n)
        cp_s.wait()
        cp_d.wait()
        ones = jnp.ones((_L,), jnp.float32)

        def body(i, c):
            s = src_v[pl.ds(i * _L, _L)]
            d = dst_v[pl.ds(i * _L, _L)]
            plsc.addupdate_scatter(acc_o, [s], ones)
            plsc.addupdate_scatter(acc_i, [d], ones)
            return c

        lax.fori_loop(0, epw // _L, body, 0, unroll=8)
        pltpu.sync_copy(acc_o, out_o_hbm.at[wid])
        pltpu.sync_copy(acc_i, out_i_hbm.at[wid])

    return degrees


def _make_sc_bpass(n, e):
    epw = e // _NW

    @functools.partial(
        pl.kernel,
        mesh=_MESH,
        out_type=jax.ShapeDtypeStruct((_NW, n), jnp.float32),
        scratch_types=[
            pltpu.VMEM((epw,), jnp.int32),
            pltpu.VMEM((epw,), jnp.int32),
            pltpu.VMEM((n,), jnp.float32),
            pltpu.VMEM((n,), jnp.float32),
            pltpu.SemaphoreType.DMA,
        ],
        compiler_params=_SC_PARAMS,
    )
    def bpass(src_hbm, dst_hbm, p_hbm, out_hbm, src_v, dst_v, p_v, acc, sem):
        wid = _worker_id()
        base = wid * epw
        cp_s = pltpu.async_copy(src_hbm.at[pl.ds(base, epw)], src_v, sem)
        cp_d = pltpu.async_copy(dst_hbm.at[pl.ds(base, epw)], dst_v, sem)
        cp_p = pltpu.async_copy(p_hbm, p_v, sem)
        _zero_vmem(acc, n)
        cp_s.wait()
        cp_d.wait()
        cp_p.wait()

        def body(i, c):
            d = dst_v[pl.ds(i * _L, _L)]
            vals = plsc.load_gather(p_v, [d])
            s = src_v[pl.ds(i * _L, _L)]
            plsc.addupdate_scatter(acc, [s], vals)
            return c

        lax.fori_loop(0, epw // _L, body, 0, unroll=8)
        pltpu.sync_copy(acc, out_hbm.at[wid])

    return bpass


def _tc_norms_body(po_ref, pi_ref, ns_ref, nd_ref, p_ref, inv_n):
    deg_o = jnp.sum(po_ref[...], axis=0, keepdims=True)
    deg_i = jnp.sum(pi_ref[...], axis=0, keepdims=True)
    ns = lax.rsqrt(jnp.maximum(deg_o, 1.0))
    nd = lax.rsqrt(jnp.maximum(deg_i, 1.0))
    ns_ref[...] = ns
    nd_ref[...] = nd
    p_ref[...] = nd * inv_n


def _tc_mid_body(part_ref, ns_ref, nd_ref, p_ref, sig_ref):
    a = jnp.sum(part_ref[...], axis=0, keepdims=True)
    v = ns_ref[...] * a
    sig_ref[...] = jnp.sum(v, axis=(0, 1), keepdims=True)
    p_ref[...] = nd_ref[...] * v


def _leaky(v):
    return jnp.where(v >= 0, v, 0.01 * v)


def _tc_head_body(
    part_ref, ns_ref, x_ref,
    w1_ref, b1_ref, w2_ref, b2_ref, w3_ref, b3_ref,
    wc1_ref, bc1_ref, wc2_ref, bc2_ref, wc3_ref, bc3_ref,
    s2_ref, s3_ref, y_ref,
):
    a = jnp.sum(part_ref[...], axis=0, keepdims=True)
    v1 = ns_ref[...] * a
    r = jnp.dot(v1, x_ref[...], preferred_element_type=jnp.float32)
    r = jnp.dot(r, w1_ref[...], preferred_element_type=jnp.float32) + s2_ref[...] * b1_ref[...]
    r = jnp.dot(r, w2_ref[...], preferred_element_type=jnp.float32) + s3_ref[...] * b2_ref[...]
    g = jnp.dot(r, w3_ref[...], preferred_element_type=jnp.float32) + b3_ref[...]
    y = _leaky(jnp.dot(g, wc1_ref[...], preferred_element_type=jnp.float32) + bc1_ref[...])
    y = _leaky(jnp.dot(y, wc2_ref[...], preferred_element_type=jnp.float32) + bc2_ref[...])
    y = _leaky(jnp.dot(y, wc3_ref[...], preferred_element_type=jnp.float32) + bc3_ref[...])
    y_ref[...] = y


def kernel(x, edge_index, W1, b1, W2, b2, W3, b3, Wc1, bc1, Wc2, bc2, Wc3, bc3):
    n, d_in = x.shape
    e = edge_index.shape[1]
    src = edge_index[0]
    dst = edge_index[1]

    sc_degrees = _make_sc_degrees(n, e)
    sc_bpass = _make_sc_bpass(n, e)

    po, pi = sc_degrees(src, dst)

    f32 = jnp.float32
    norms = pl.pallas_call(
        functools.partial(_tc_norms_body, inv_n=1.0 / n),
        out_shape=[
            jax.ShapeDtypeStruct((1, n), f32),
            jax.ShapeDtypeStruct((1, n), f32),
            jax.ShapeDtypeStruct((1, n), f32),
        ],
    )
    ns, nd, p = norms(po, pi)

    mid = pl.pallas_call(
        _tc_mid_body,
        out_shape=[
            jax.ShapeDtypeStruct((1, n), f32),
            jax.ShapeDtypeStruct((1, 1), f32),
        ],
    )

    part = sc_bpass(src, dst, p.reshape(n))
    p, s3 = mid(part, ns, nd)
    part = sc_bpass(src, dst, p.reshape(n))
    p, s2 = mid(part, ns, nd)
    part = sc_bpass(src, dst, p.reshape(n))

    head = pl.pallas_call(
        _tc_head_body,
        out_shape=jax.ShapeDtypeStruct((1, Wc3.shape[1]), f32),
    )
    y = head(
        part, ns, x,
        W1, b1.reshape(1, -1), W2, b2.reshape(1, -1), W3, b3.reshape(1, -1),
        Wc1, bc1.reshape(1, -1), Wc2, bc2.reshape(1, -1), Wc3, bc3.reshape(1, -1),
        s2, s3,
    )
    return y


# speedup vs baseline: 52.4863x; 1.0085x over previous
import functools

import jax
import jax.numpy as jnp
from jax import lax
from jax.experimental import pallas as pl
from jax.experimental.pallas import tpu as pltpu
from jax.experimental.pallas import tpu_sc as plsc

_NC = 2
_NS = 16
_NW = _NC * _NS
_L = 16

_MESH = plsc.VectorSubcoreMesh(core_axis_name="c", subcore_axis_name="s")
_SC_PARAMS = pltpu.CompilerParams(needs_layout_passes=False)


def _worker_id():
    return lax.axis_index("s") * _NC + lax.axis_index("c")


def _zero_vmem(ref, n):
    z = jnp.zeros((_L,), jnp.float32)

    def body(i, c):
        ref[pl.ds(i * _L, _L)] = z
        return c

    lax.fori_loop(0, n // _L, body, 0, unroll=8)


def _store_partial_chunks(acc, out3_hbm, wid, ch, sem):
    cps = [
        pltpu.async_copy(acc.at[pl.ds(j * ch, ch)], out3_hbm.at[j, wid], sem)
        for j in range(_NS)
    ]
    for cp in cps:
        cp.wait()


def _make_sc_degrees(npad, e):
    epw = e // _NW
    ch = npad // _NS

    @functools.partial(
        pl.kernel,
        mesh=_MESH,
        out_type=[
            jax.ShapeDtypeStruct((_NS, _NW, ch), jnp.float32),
            jax.ShapeDtypeStruct((_NS, _NW, ch), jnp.float32),
        ],
        scratch_types=[
            pltpu.VMEM((epw,), jnp.int32),
            pltpu.VMEM((epw,), jnp.int32),
            pltpu.VMEM((npad,), jnp.float32),
            pltpu.VMEM((npad,), jnp.float32),
            pltpu.SemaphoreType.DMA,
        ],
        compiler_params=_SC_PARAMS,
    )
    def degrees(src_hbm, dst_hbm, out_o_hbm, out_i_hbm, src_v, dst_v, acc_o, acc_i, sem):
        wid = _worker_id()
        base = wid * epw
        cp_s = pltpu.async_copy(src_hbm.at[pl.ds(base, epw)], src_v, sem)
        cp_d = pltpu.async_copy(dst_hbm.at[pl.ds(base, epw)], dst_v, sem)
        _zero_vmem(acc_o, npad)
        _zero_vmem(acc_i, npad)
        cp_s.wait()
        cp_d.wait()
        ones = jnp.ones((_L,), jnp.float32)

        def body(i, c):
            s = src_v[pl.ds(i * _L, _L)]
            d = dst_v[pl.ds(i * _L, _L)]
            plsc.addupdate_scatter(acc_o, [s], ones)
            plsc.addupdate_scatter(acc_i, [d], ones)
            return c

        lax.fori_loop(0, epw // _L, body, 0, unroll=8)
        _store_partial_chunks(acc_o, out_o_hbm, wid, ch, sem)
        _store_partial_chunks(acc_i, out_i_hbm, wid, ch, sem)

    return degrees


def _make_sc_bpass1(npad, e):
    epw = e // _NW
    ch = npad // _NS

    @functools.partial(
        pl.kernel,
        mesh=_MESH,
        out_type=jax.ShapeDtypeStruct((_NS, _NW, ch), jnp.float32),
        scratch_types=[
            pltpu.VMEM((epw,), jnp.int32),
            pltpu.VMEM((epw,), jnp.int32),
            pltpu.VMEM((npad,), jnp.float32),
            pltpu.VMEM((npad,), jnp.float32),
            pltpu.SemaphoreType.DMA,
        ],
        compiler_params=_SC_PARAMS,
    )
    def bpass1(src_hbm, dst_hbm, p_hbm, out_hbm, src_v, dst_v, p_v, acc, sem):
        wid = _worker_id()
        base = wid * epw
        cp_s = pltpu.async_copy(src_hbm.at[pl.ds(base, epw)], src_v, sem)
        cp_d = pltpu.async_copy(dst_hbm.at[pl.ds(base, epw)], dst_v, sem)
        cp_p = pltpu.async_copy(p_hbm.at[0], p_v, sem)
        _zero_vmem(acc, npad)
        cp_s.wait()
        cp_d.wait()
        cp_p.wait()

        def body(i, c):
            d = dst_v[pl.ds(i * _L, _L)]
            vals = plsc.load_gather(p_v, [d])
            s = src_v[pl.ds(i * _L, _L)]
            plsc.addupdate_scatter(acc, [s], vals)
            return c

        lax.fori_loop(0, epw // _L, body, 0, unroll=8)
        _store_partial_chunks(acc, out_hbm, wid, ch, sem)

    return bpass1


def _make_sc_bpass_r(npad, e):
    epw = e // _NW
    ch = npad // _NS

    @functools.partial(
        pl.kernel,
        mesh=_MESH,
        out_type=jax.ShapeDtypeStruct((_NS, _NW, ch), jnp.float32),
        scratch_types=[
            pltpu.VMEM((epw,), jnp.int32),
            pltpu.VMEM((epw,), jnp.int32),
            pltpu.VMEM((npad,), jnp.float32),
            pltpu.VMEM((npad,), jnp.float32),
            pltpu.VMEM((_NW, ch), jnp.float32),
            pltpu.VMEM((ch,), jnp.float32),
            pltpu.VMEM((ch,), jnp.float32),
            pltpu.VMEM_SHARED((_NS, ch), jnp.float32),
            pltpu.SemaphoreType.DMA,
            pltpu.SemaphoreType.DMA,
            pltpu.SemaphoreType.DMA,
        ],
        compiler_params=_SC_PARAMS,
    )
    def bpass_r(src_hbm, dst_hbm, part_hbm, q_hbm, out_hbm,
                src_v, dst_v, p_v, acc, rbuf, qbuf, pchunk, sp_p,
                sem_e, sem_r, sem_p):
        sid = lax.axis_index("s")
        wid = _worker_id()
        base = wid * epw
        cs = sid * ch
        cp_s = pltpu.async_copy(src_hbm.at[pl.ds(base, epw)], src_v, sem_e)
        cp_d = pltpu.async_copy(dst_hbm.at[pl.ds(base, epw)], dst_v, sem_e)
        cp_r = pltpu.async_copy(part_hbm.at[sid], rbuf, sem_r)
        cp_q = pltpu.async_copy(q_hbm.at[0, pl.ds(cs, ch)], qbuf, sem_r)
        _zero_vmem(acc, npad)
        cp_r.wait()
        cp_q.wait()

        def red_body(c, carry):
            sl = pl.ds(c * _L, _L)
            val = rbuf[0, sl]
            for r in range(1, _NW):
                val = val + rbuf[r, sl]
            pchunk[sl] = val * qbuf[sl]
            return carry

        lax.fori_loop(0, ch // _L, red_body, 0)
        pltpu.sync_copy(pchunk, sp_p.at[sid])
        plsc.subcore_barrier()
        cps = [
            pltpu.async_copy(sp_p.at[j], p_v.at[pl.ds(j * ch, ch)], sem_p)
            for j in range(_NS)
        ]
        for cp in cps:
            cp.wait()
        cp_s.wait()
        cp_d.wait()

        def body(i, c):
            d = dst_v[pl.ds(i * _L, _L)]
            vals = plsc.load_gather(p_v, [d])
            s = src_v[pl.ds(i * _L, _L)]
            plsc.addupdate_scatter(acc, [s], vals)
            return c

        lax.fori_loop(0, epw // _L, body, 0, unroll=8)
        _store_partial_chunks(acc, out_hbm, wid, ch, sem_e)

    return bpass_r


def _tc_norms_body(po_ref, pi_ref, ns_ref, q_ref, p_ref, inv_n, npad):
    deg_o = jnp.sum(po_ref[...], axis=1).reshape(1, npad)
    deg_i = jnp.sum(pi_ref[...], axis=1).reshape(1, npad)
    ns = lax.rsqrt(jnp.maximum(deg_o, 1.0))
    nd = lax.rsqrt(jnp.maximum(deg_i, 1.0))
    ns_ref[...] = ns
    q_ref[...] = ns * nd
    p_ref[...] = nd * inv_n


def _leaky(v):
    return jnp.where(v >= 0, v, 0.01 * v)


def _tc_head_body(
    p1_ref, p2_ref, p3_ref, ns_ref, x_ref,
    w1_ref, b1_ref, w2_ref, b2_ref, w3_ref, b3_ref,
    wc1_ref, bc1_ref, wc2_ref, bc2_ref, wc3_ref, bc3_ref,
    y_ref, *, n, npad,
):
    ns = ns_ref[...]
    v3 = ns * jnp.sum(p1_ref[...], axis=1).reshape(1, npad)
    v2 = ns * jnp.sum(p2_ref[...], axis=1).reshape(1, npad)
    v1 = ns * jnp.sum(p3_ref[...], axis=1).reshape(1, npad)
    s3 = jnp.sum(v3, axis=1, keepdims=True)
    s2 = jnp.sum(v2, axis=1, keepdims=True)
    r = jnp.dot(v1[:, :n], x_ref[...], preferred_element_type=jnp.float32)
    r = jnp.dot(r, w1_ref[...], preferred_element_type=jnp.float32) + s2 * b1_ref[...]
    r = jnp.dot(r, w2_ref[...], preferred_element_type=jnp.float32) + s3 * b2_ref[...]
    g = jnp.dot(r, w3_ref[...], preferred_element_type=jnp.float32) + b3_ref[...]
    y = _leaky(jnp.dot(g, wc1_ref[...], preferred_element_type=jnp.float32) + bc1_ref[...])
    y = _leaky(jnp.dot(y, wc2_ref[...], preferred_element_type=jnp.float32) + bc2_ref[...])
    y = _leaky(jnp.dot(y, wc3_ref[...], preferred_element_type=jnp.float32) + bc3_ref[...])
    y_ref[...] = y


def kernel(x, edge_index, W1, b1, W2, b2, W3, b3, Wc1, bc1, Wc2, bc2, Wc3, bc3):
    n, _ = x.shape
    e = edge_index.shape[1]
    grain = _NS * _L
    npad = ((n + grain - 1) // grain) * grain
    src = edge_index[0]
    dst = edge_index[1]

    sc_degrees = _make_sc_degrees(npad, e)
    sc_bpass1 = _make_sc_bpass1(npad, e)
    sc_bpass_r = _make_sc_bpass_r(npad, e)

    po, pi = sc_degrees(src, dst)

    f32 = jnp.float32
    norms = pl.pallas_call(
        functools.partial(_tc_norms_body, inv_n=1.0 / n, npad=npad),
        out_shape=[
            jax.ShapeDtypeStruct((1, npad), f32),
            jax.ShapeDtypeStruct((1, npad), f32),
            jax.ShapeDtypeStruct((1, npad), f32),
        ],
    )
    ns, q, p0 = norms(po, pi)

    part1 = sc_bpass1(src, dst, p0)
    part2 = sc_bpass_r(src, dst, part1, q)
    part3 = sc_bpass_r(src, dst, part2, q)

    head = pl.pallas_call(
        functools.partial(_tc_head_body, n=n, npad=npad),
        out_shape=jax.ShapeDtypeStruct((1, Wc3.shape[1]), f32),
    )
    y = head(
        part1, part2, part3, ns, x,
        W1, b1.reshape(1, -1), W2, b2.reshape(1, -1), W3, b3.reshape(1, -1),
        Wc1, bc1.reshape(1, -1), Wc2, bc2.reshape(1, -1), Wc3, bc3.reshape(1, -1),
    )
    return y


# speedup vs baseline: 59.1784x; 1.1275x over previous
import functools

import jax
import jax.numpy as jnp
from jax import lax
from jax.experimental import pallas as pl
from jax.experimental.pallas import tpu as pltpu
from jax.experimental.pallas import tpu_sc as plsc

_NC = 2
_NS = 16
_NW = _NC * _NS
_L = 16

_MESH = plsc.VectorSubcoreMesh(core_axis_name="c", subcore_axis_name="s")
_SC_PARAMS = pltpu.CompilerParams(needs_layout_passes=False)


def _worker_id():
    return lax.axis_index("s") * _NC + lax.axis_index("c")


def _zero_vmem(ref, n):
    z = jnp.zeros((_L,), jnp.float32)

    def body(i, c):
        ref[pl.ds(i * _L, _L)] = z
        return c

    lax.fori_loop(0, n // _L, body, 0, unroll=8)


def _store_partial_chunks(acc, out3_hbm, wid, ch, sem):
    cps = [
        pltpu.async_copy(acc.at[pl.ds(j * ch, ch)], out3_hbm.at[j, wid], sem)
        for j in range(_NS)
    ]
    for cp in cps:
        cp.wait()


def _make_sc_degrees(npad, e):
    epw = e // _NW
    ch = npad // _NS

    @functools.partial(
        pl.kernel,
        mesh=_MESH,
        out_type=[
            jax.ShapeDtypeStruct((_NS, _NW, ch), jnp.float32),
            jax.ShapeDtypeStruct((_NS, _NW, ch), jnp.float32),
        ],
        scratch_types=[
            pltpu.VMEM((epw,), jnp.int32),
            pltpu.VMEM((epw,), jnp.int32),
            pltpu.VMEM((npad,), jnp.float32),
            pltpu.VMEM((npad,), jnp.float32),
            pltpu.SemaphoreType.DMA,
        ],
        compiler_params=_SC_PARAMS,
    )
    def degrees(ei_hbm, out_o_hbm, out_i_hbm, src_v, dst_v, acc_o, acc_i, sem):
        wid = _worker_id()
        base = wid * epw
        cp_s = pltpu.async_copy(ei_hbm.at[pl.ds(base, epw)], src_v, sem)
        cp_d = pltpu.async_copy(ei_hbm.at[pl.ds(e + base, epw)], dst_v, sem)
        _zero_vmem(acc_o, npad)
        _zero_vmem(acc_i, npad)
        cp_s.wait()
        cp_d.wait()
        ones = jnp.ones((_L,), jnp.float32)

        def body(i, c):
            s = src_v[pl.ds(i * _L, _L)]
            d = dst_v[pl.ds(i * _L, _L)]
            plsc.addupdate_scatter(acc_o, [s], ones)
            plsc.addupdate_scatter(acc_i, [d], ones)
            return c

        lax.fori_loop(0, epw // _L, body, 0, unroll=8)
        _store_partial_chunks(acc_o, out_o_hbm, wid, ch, sem)
        _store_partial_chunks(acc_i, out_i_hbm, wid, ch, sem)

    return degrees


def _make_sc_bpass1(npad, e):
    epw = e // _NW
    ch = npad // _NS

    @functools.partial(
        pl.kernel,
        mesh=_MESH,
        out_type=jax.ShapeDtypeStruct((_NS, _NW, ch), jnp.float32),
        scratch_types=[
            pltpu.VMEM((epw,), jnp.int32),
            pltpu.VMEM((epw,), jnp.int32),
            pltpu.VMEM((npad,), jnp.float32),
            pltpu.VMEM((npad,), jnp.float32),
            pltpu.SemaphoreType.DMA,
        ],
        compiler_params=_SC_PARAMS,
    )
    def bpass1(ei_hbm, p_hbm, out_hbm, src_v, dst_v, p_v, acc, sem):
        wid = _worker_id()
        base = wid * epw
        cp_s = pltpu.async_copy(ei_hbm.at[pl.ds(base, epw)], src_v, sem)
        cp_d = pltpu.async_copy(ei_hbm.at[pl.ds(e + base, epw)], dst_v, sem)
        cp_p = pltpu.async_copy(p_hbm.at[0], p_v, sem)
        _zero_vmem(acc, npad)
        cp_s.wait()
        cp_d.wait()
        cp_p.wait()

        def body(i, c):
            d = dst_v[pl.ds(i * _L, _L)]
            vals = plsc.load_gather(p_v, [d])
            s = src_v[pl.ds(i * _L, _L)]
            plsc.addupdate_scatter(acc, [s], vals)
            return c

        lax.fori_loop(0, epw // _L, body, 0, unroll=8)
        _store_partial_chunks(acc, out_hbm, wid, ch, sem)

    return bpass1


def _make_sc_bpass_r(npad, e):
    epw = e // _NW
    ch = npad // _NS

    @functools.partial(
        pl.kernel,
        mesh=_MESH,
        out_type=jax.ShapeDtypeStruct((_NS, _NW, ch), jnp.float32),
        scratch_types=[
            pltpu.VMEM((epw,), jnp.int32),
            pltpu.VMEM((epw,), jnp.int32),
            pltpu.VMEM((npad,), jnp.float32),
            pltpu.VMEM((npad,), jnp.float32),
            pltpu.VMEM((_NW, ch), jnp.float32),
            pltpu.VMEM((ch,), jnp.float32),
            pltpu.VMEM((ch,), jnp.float32),
            pltpu.VMEM_SHARED((_NS, ch), jnp.float32),
            pltpu.SemaphoreType.DMA,
            pltpu.SemaphoreType.DMA,
            pltpu.SemaphoreType.DMA,
        ],
        compiler_params=_SC_PARAMS,
    )
    def bpass_r(ei_hbm, part_hbm, q_hbm, out_hbm,
                src_v, dst_v, p_v, acc, rbuf, qbuf, pchunk, sp_p,
                sem_e, sem_r, sem_p):
        sid = lax.axis_index("s")
        wid = _worker_id()
        base = wid * epw
        cs = sid * ch
        cp_s = pltpu.async_copy(ei_hbm.at[pl.ds(base, epw)], src_v, sem_e)
        cp_d = pltpu.async_copy(ei_hbm.at[pl.ds(e + base, epw)], dst_v, sem_e)
        cp_r = pltpu.async_copy(part_hbm.at[sid], rbuf, sem_r)
        cp_q = pltpu.async_copy(q_hbm.at[0, pl.ds(cs, ch)], qbuf, sem_r)
        _zero_vmem(acc, npad)
        cp_r.wait()
        cp_q.wait()

        def red_body(c, carry):
            sl = pl.ds(c * _L, _L)
            val = rbuf[0, sl]
            for r in range(1, _NW):
                val = val + rbuf[r, sl]
            pchunk[sl] = val * qbuf[sl]
            return carry

        lax.fori_loop(0, ch // _L, red_body, 0)
        pltpu.sync_copy(pchunk, sp_p.at[sid])
        plsc.subcore_barrier()
        cps = [
            pltpu.async_copy(sp_p.at[j], p_v.at[pl.ds(j * ch, ch)], sem_p)
            for j in range(_NS)
        ]
        for cp in cps:
            cp.wait()
        cp_s.wait()
        cp_d.wait()

        def body(i, c):
            d = dst_v[pl.ds(i * _L, _L)]
            vals = plsc.load_gather(p_v, [d])
            s = src_v[pl.ds(i * _L, _L)]
            plsc.addupdate_scatter(acc, [s], vals)
            return c

        lax.fori_loop(0, epw // _L, body, 0, unroll=8)
        _store_partial_chunks(acc, out_hbm, wid, ch, sem_e)

    return bpass_r


def _tc_norms_body(po_ref, pi_ref, ns_ref, q_ref, p_ref, inv_n, npad):
    deg_o = jnp.sum(po_ref[...], axis=1).reshape(1, npad)
    deg_i = jnp.sum(pi_ref[...], axis=1).reshape(1, npad)
    ns = lax.rsqrt(jnp.maximum(deg_o, 1.0))
    nd = lax.rsqrt(jnp.maximum(deg_i, 1.0))
    ns_ref[...] = ns
    q_ref[...] = ns * nd
    p_ref[...] = nd * inv_n


def _leaky(v):
    return jnp.where(v >= 0, v, 0.01 * v)


def _tc_head_body(
    p1_ref, p2_ref, p3_ref, ns_ref, x_ref,
    w1_ref, b1_ref, w2_ref, b2_ref, w3_ref, b3_ref,
    wc1_ref, bc1_ref, wc2_ref, bc2_ref, wc3_ref, bc3_ref,
    y_ref, *, n, npad,
):
    ns = ns_ref[...]
    v3 = ns * jnp.sum(p1_ref[...], axis=1).reshape(1, npad)
    v2 = ns * jnp.sum(p2_ref[...], axis=1).reshape(1, npad)
    v1 = ns * jnp.sum(p3_ref[...], axis=1).reshape(1, npad)
    s3 = jnp.sum(v3, axis=1, keepdims=True)
    s2 = jnp.sum(v2, axis=1, keepdims=True)
    r = jnp.dot(v1[:, :n], x_ref[...], preferred_element_type=jnp.float32)
    r = jnp.dot(r, w1_ref[...], preferred_element_type=jnp.float32) + s2 * b1_ref[...]
    r = jnp.dot(r, w2_ref[...], preferred_element_type=jnp.float32) + s3 * b2_ref[...]
    g = jnp.dot(r, w3_ref[...], preferred_element_type=jnp.float32) + b3_ref[...]
    y = _leaky(jnp.dot(g, wc1_ref[...], preferred_element_type=jnp.float32) + bc1_ref[...])
    y = _leaky(jnp.dot(y, wc2_ref[...], preferred_element_type=jnp.float32) + bc2_ref[...])
    y = _leaky(jnp.dot(y, wc3_ref[...], preferred_element_type=jnp.float32) + bc3_ref[...])
    y_ref[...] = y


def kernel(x, edge_index, W1, b1, W2, b2, W3, b3, Wc1, bc1, Wc2, bc2, Wc3, bc3):
    n, _ = x.shape
    e = edge_index.shape[1]
    grain = _NS * _L
    npad = ((n + grain - 1) // grain) * grain

    sc_degrees = _make_sc_degrees(npad, e)
    sc_bpass1 = _make_sc_bpass1(npad, e)
    sc_bpass_r = _make_sc_bpass_r(npad, e)

    ei_flat = edge_index.reshape(2 * e)
    po, pi = sc_degrees(ei_flat)

    f32 = jnp.float32
    norms = pl.pallas_call(
        functools.partial(_tc_norms_body, inv_n=1.0 / n, npad=npad),
        out_shape=[
            jax.ShapeDtypeStruct((1, npad), f32),
            jax.ShapeDtypeStruct((1, npad), f32),
            jax.ShapeDtypeStruct((1, npad), f32),
        ],
    )
    ns, q, p0 = norms(po, pi)

    part1 = sc_bpass1(ei_flat, p0)
    part2 = sc_bpass_r(ei_flat, part1, q)
    part3 = sc_bpass_r(ei_flat, part2, q)

    head = pl.pallas_call(
        functools.partial(_tc_head_body, n=n, npad=npad),
        out_shape=jax.ShapeDtypeStruct((1, Wc3.shape[1]), f32),
    )
    y = head(
        part1, part2, part3, ns, x,
        W1, b1.reshape(1, -1), W2, b2.reshape(1, -1), W3, b3.reshape(1, -1),
        Wc1, bc1.reshape(1, -1), Wc2, bc2.reshape(1, -1), Wc3, bc3.reshape(1, -1),
    )
    return y


# speedup vs baseline: 71.1465x; 1.2022x over previous
import functools

import jax
import jax.numpy as jnp
from jax import lax
from jax.experimental import pallas as pl
from jax.experimental.pallas import tpu as pltpu
from jax.experimental.pallas import tpu_sc as plsc

_NC = 2
_NS = 16
_NW = _NC * _NS
_L = 16

_MESH = plsc.VectorSubcoreMesh(core_axis_name="c", subcore_axis_name="s")
_SC_PARAMS = pltpu.CompilerParams(needs_layout_passes=False)


def _worker_id():
    return lax.axis_index("s") * _NC + lax.axis_index("c")


def _zero_vmem(ref, n):
    z = jnp.zeros((_L,), jnp.float32)

    @plsc.parallel_loop(0, n // _L, unroll=8)
    def _(i):
        ref[pl.ds(i * _L, _L)] = z


def _store_partial_chunks(acc, out3_hbm, wid, ch, sem):
    cps = [
        pltpu.async_copy(acc.at[pl.ds(j * ch, ch)], out3_hbm.at[j, wid], sem)
        for j in range(_NS)
    ]
    for cp in cps:
        cp.wait()


def _make_sc_degrees(npad, e):
    epw = e // _NW
    ch = npad // _NS

    @functools.partial(
        pl.kernel,
        mesh=_MESH,
        out_type=[
            jax.ShapeDtypeStruct((_NS, _NW, ch), jnp.float32),
            jax.ShapeDtypeStruct((_NS, _NW, ch), jnp.float32),
        ],
        scratch_types=[
            pltpu.VMEM((epw,), jnp.int32),
            pltpu.VMEM((epw,), jnp.int32),
            pltpu.VMEM((npad,), jnp.float32),
            pltpu.VMEM((npad,), jnp.float32),
            pltpu.SemaphoreType.DMA,
        ],
        compiler_params=_SC_PARAMS,
    )
    def degrees(ei_hbm, out_o_hbm, out_i_hbm, src_v, dst_v, acc_o, acc_i, sem):
        wid = _worker_id()
        base = wid * epw
        cp_s = pltpu.async_copy(ei_hbm.at[pl.ds(base, epw)], src_v, sem)
        cp_d = pltpu.async_copy(ei_hbm.at[pl.ds(e + base, epw)], dst_v, sem)
        _zero_vmem(acc_o, npad)
        _zero_vmem(acc_i, npad)
        cp_s.wait()
        cp_d.wait()
        ones = jnp.ones((_L,), jnp.float32)

        @plsc.parallel_loop(0, epw // _L, unroll=8)
        def _(i):
            s = src_v[pl.ds(i * _L, _L)]
            d = dst_v[pl.ds(i * _L, _L)]
            plsc.addupdate_scatter(acc_o, [s], ones)
            plsc.addupdate_scatter(acc_i, [d], ones)
        _store_partial_chunks(acc_o, out_o_hbm, wid, ch, sem)
        _store_partial_chunks(acc_i, out_i_hbm, wid, ch, sem)

    return degrees


def _make_sc_bpass1(npad, e):
    epw = e // _NW
    ch = npad // _NS

    @functools.partial(
        pl.kernel,
        mesh=_MESH,
        out_type=jax.ShapeDtypeStruct((_NS, _NW, ch), jnp.float32),
        scratch_types=[
            pltpu.VMEM((epw,), jnp.int32),
            pltpu.VMEM((epw,), jnp.int32),
            pltpu.VMEM((npad,), jnp.float32),
            pltpu.VMEM((npad,), jnp.float32),
            pltpu.SemaphoreType.DMA,
        ],
        compiler_params=_SC_PARAMS,
    )
    def bpass1(ei_hbm, p_hbm, out_hbm, src_v, dst_v, p_v, acc, sem):
        wid = _worker_id()
        base = wid * epw
        cp_s = pltpu.async_copy(ei_hbm.at[pl.ds(base, epw)], src_v, sem)
        cp_d = pltpu.async_copy(ei_hbm.at[pl.ds(e + base, epw)], dst_v, sem)
        cp_p = pltpu.async_copy(p_hbm.at[0], p_v, sem)
        _zero_vmem(acc, npad)
        cp_s.wait()
        cp_d.wait()
        cp_p.wait()

        @plsc.parallel_loop(0, epw // _L, unroll=8)
        def _(i):
            d = dst_v[pl.ds(i * _L, _L)]
            vals = plsc.load_gather(p_v, [d])
            s = src_v[pl.ds(i * _L, _L)]
            plsc.addupdate_scatter(acc, [s], vals)
        _store_partial_chunks(acc, out_hbm, wid, ch, sem)

    return bpass1


def _make_sc_bpass_r(npad, e):
    epw = e // _NW
    ch = npad // _NS

    @functools.partial(
        pl.kernel,
        mesh=_MESH,
        out_type=jax.ShapeDtypeStruct((_NS, _NW, ch), jnp.float32),
        scratch_types=[
            pltpu.VMEM((epw,), jnp.int32),
            pltpu.VMEM((epw,), jnp.int32),
            pltpu.VMEM((npad,), jnp.float32),
            pltpu.VMEM((npad,), jnp.float32),
            pltpu.VMEM((_NW, ch), jnp.float32),
            pltpu.VMEM((ch,), jnp.float32),
            pltpu.VMEM((ch,), jnp.float32),
            pltpu.VMEM_SHARED((_NS, ch), jnp.float32),
            pltpu.SemaphoreType.DMA,
            pltpu.SemaphoreType.DMA,
            pltpu.SemaphoreType.DMA,
        ],
        compiler_params=_SC_PARAMS,
    )
    def bpass_r(ei_hbm, part_hbm, q_hbm, out_hbm,
                src_v, dst_v, p_v, acc, rbuf, qbuf, pchunk, sp_p,
                sem_e, sem_r, sem_p):
        sid = lax.axis_index("s")
        wid = _worker_id()
        base = wid * epw
        cs = sid * ch
        cp_s = pltpu.async_copy(ei_hbm.at[pl.ds(base, epw)], src_v, sem_e)
        cp_d = pltpu.async_copy(ei_hbm.at[pl.ds(e + base, epw)], dst_v, sem_e)
        cp_r = pltpu.async_copy(part_hbm.at[sid], rbuf, sem_r)
        cp_q = pltpu.async_copy(q_hbm.at[0, pl.ds(cs, ch)], qbuf, sem_r)
        _zero_vmem(acc, npad)
        cp_r.wait()
        cp_q.wait()

        def red_body(c, carry):
            sl = pl.ds(c * _L, _L)
            val = rbuf[0, sl]
            for r in range(1, _NW):
                val = val + rbuf[r, sl]
            pchunk[sl] = val * qbuf[sl]
            return carry

        lax.fori_loop(0, ch // _L, red_body, 0)
        pltpu.sync_copy(pchunk, sp_p.at[sid])
        plsc.subcore_barrier()
        cps = [
            pltpu.async_copy(sp_p.at[j], p_v.at[pl.ds(j * ch, ch)], sem_p)
            for j in range(_NS)
        ]
        for cp in cps:
            cp.wait()
        cp_s.wait()
        cp_d.wait()

        @plsc.parallel_loop(0, epw // _L, unroll=8)
        def _(i):
            d = dst_v[pl.ds(i * _L, _L)]
            vals = plsc.load_gather(p_v, [d])
            s = src_v[pl.ds(i * _L, _L)]
            plsc.addupdate_scatter(acc, [s], vals)
        _store_partial_chunks(acc, out_hbm, wid, ch, sem_e)

    return bpass_r


def _tc_norms_body(po_ref, pi_ref, ns_ref, q_ref, p_ref, inv_n, npad):
    deg_o = jnp.sum(po_ref[...], axis=1).reshape(1, npad)
    deg_i = jnp.sum(pi_ref[...], axis=1).reshape(1, npad)
    ns = lax.rsqrt(jnp.maximum(deg_o, 1.0))
    nd = lax.rsqrt(jnp.maximum(deg_i, 1.0))
    ns_ref[...] = ns
    q_ref[...] = ns * nd
    p_ref[...] = nd * inv_n


def _leaky(v):
    return jnp.where(v >= 0, v, 0.01 * v)


def _tc_head_body(
    p1_ref, p2_ref, p3_ref, ns_ref, x_ref,
    w1_ref, b1_ref, w2_ref, b2_ref, w3_ref, b3_ref,
    wc1_ref, bc1_ref, wc2_ref, bc2_ref, wc3_ref, bc3_ref,
    y_ref, *, n, npad,
):
    ns = ns_ref[...]
    v3 = ns * jnp.sum(p1_ref[...], axis=1).reshape(1, npad)
    v2 = ns * jnp.sum(p2_ref[...], axis=1).reshape(1, npad)
    v1 = ns * jnp.sum(p3_ref[...], axis=1).reshape(1, npad)
    s3 = jnp.sum(v3, axis=1, keepdims=True)
    s2 = jnp.sum(v2, axis=1, keepdims=True)
    r = jnp.dot(v1[:, :n], x_ref[...], preferred_element_type=jnp.float32)
    r = jnp.dot(r, w1_ref[...], preferred_element_type=jnp.float32) + s2 * b1_ref[...]
    r = jnp.dot(r, w2_ref[...], preferred_element_type=jnp.float32) + s3 * b2_ref[...]
    g = jnp.dot(r, w3_ref[...], preferred_element_type=jnp.float32) + b3_ref[...]
    y = _leaky(jnp.dot(g, wc1_ref[...], preferred_element_type=jnp.float32) + bc1_ref[...])
    y = _leaky(jnp.dot(y, wc2_ref[...], preferred_element_type=jnp.float32) + bc2_ref[...])
    y = _leaky(jnp.dot(y, wc3_ref[...], preferred_element_type=jnp.float32) + bc3_ref[...])
    y_ref[...] = y


def kernel(x, edge_index, W1, b1, W2, b2, W3, b3, Wc1, bc1, Wc2, bc2, Wc3, bc3):
    n, _ = x.shape
    e = edge_index.shape[1]
    grain = _NS * _L
    npad = ((n + grain - 1) // grain) * grain

    sc_degrees = _make_sc_degrees(npad, e)
    sc_bpass1 = _make_sc_bpass1(npad, e)
    sc_bpass_r = _make_sc_bpass_r(npad, e)

    ei_flat = edge_index.reshape(2 * e)
    po, pi = sc_degrees(ei_flat)

    f32 = jnp.float32
    norms = pl.pallas_call(
        functools.partial(_tc_norms_body, inv_n=1.0 / n, npad=npad),
        out_shape=[
            jax.ShapeDtypeStruct((1, npad), f32),
            jax.ShapeDtypeStruct((1, npad), f32),
            jax.ShapeDtypeStruct((1, npad), f32),
        ],
    )
    ns, q, p0 = norms(po, pi)

    part1 = sc_bpass1(ei_flat, p0)
    part2 = sc_bpass_r(ei_flat, part1, q)
    part3 = sc_bpass_r(ei_flat, part2, q)

    head = pl.pallas_call(
        functools.partial(_tc_head_body, n=n, npad=npad),
        out_shape=jax.ShapeDtypeStruct((1, Wc3.shape[1]), f32),
    )
    y = head(
        part1, part2, part3, ns, x,
        W1, b1.reshape(1, -1), W2, b2.reshape(1, -1), W3, b3.reshape(1, -1),
        Wc1, bc1.reshape(1, -1), Wc2, bc2.reshape(1, -1), Wc3, bc3.reshape(1, -1),
    )
    return y
